# Initial kernel scaffold; baseline (speedup 1.0000x reference)
import jax
import jax.numpy as jnp
from jax.experimental import pallas as pl


def kernel(x_init, time_grid, emb, W, u_noise, z_change):
    raise NotImplementedError("write your pallas kernel here")


# speedup vs baseline: 1.9815x; 1.9815x over previous
import jax
import jax.numpy as jnp
from jax.experimental import pallas as pl

_B, _N, _V, _D = 16, 2048, 1024, 64
_TN = 512
_G = (_B * _N) // _TN


def _body(tg_ref, emb_ref, w_ref, x_ref, u_ref, z_ref, xn_ref, pc_ref):
    x = x_ref[0]
    lane = jax.lax.broadcasted_iota(jnp.int32, (_TN, _V), 1)
    onehot = (x == lane).astype(jnp.float32)
    xemb = jnp.dot(onehot, emb_ref[...], preferred_element_type=jnp.float32)
    logits = jnp.dot(xemb, w_ref[...], preferred_element_type=jnp.float32)
    keys = logits - jnp.log(-jnp.log(u_ref[0]))
    m = jnp.max(keys, axis=1, keepdims=True)
    x1 = jnp.min(jnp.where(keys == m, lane, _V), axis=1, keepdims=True)
    t = tg_ref[0, 0]
    h = tg_ref[0, 1] - tg_ref[0, 0]
    coeff = 1.0 / (1.0 - t)
    lam = coeff * (x1 != x).astype(jnp.float32)
    p = 1.0 - jnp.exp(-h * lam)
    xn_ref[0] = jnp.where(z_ref[0] < p, x1, x)
    pc_ref[0] = p


def kernel(x_init, time_grid, emb, W, u_noise, z_change):
    xr = x_init.reshape(_G, _TN, 1)
    ur = u_noise.reshape(_G, _TN, _V)
    zr = z_change.reshape(_G, _TN, 1)
    tg = time_grid.reshape(1, 2)
    xn, pc = pl.pallas_call(
        _body,
        grid=(_G,),
        in_specs=[
            pl.BlockSpec((1, 2), lambda i: (0, 0)),
            pl.BlockSpec((_V, _D), lambda i: (0, 0)),
            pl.BlockSpec((_D, _V), lambda i: (0, 0)),
            pl.BlockSpec((1, _TN, 1), lambda i: (i, 0, 0)),
            pl.BlockSpec((1, _TN, _V), lambda i: (i, 0, 0)),
            pl.BlockSpec((1, _TN, 1), lambda i: (i, 0, 0)),
        ],
        out_specs=[
            pl.BlockSpec((1, _TN, 1), lambda i: (i, 0, 0)),
            pl.BlockSpec((1, _TN, 1), lambda i: (i, 0, 0)),
        ],
        out_shape=[
            jax.ShapeDtypeStruct((_G, _TN, 1), jnp.int32),
            jax.ShapeDtypeStruct((_G, _TN, 1), jnp.float32),
        ],
    )(tg, emb, W, xr, ur, zr)
    return xn.reshape(_B, _N), pc.reshape(_B, _N)
